# baseline (speedup 1.0000x reference)
import functools

import jax
import jax.numpy as jnp
from jax import lax
from jax.experimental import pallas as pl
from jax.experimental.pallas import tpu as pltpu
from jax.experimental.pallas import tpu_sc as plsc

VOCAB = 1000000
EMBED_DIM = 64
BATCH = 16384
HIST = 50

_NW = 32
_JT = BATCH // 128
_NU_TOTAL = HIST * _JT
_NU = _NU_TOTAL // _NW
_NPAIR = _NU // 2


def _make_gather():
    mesh = plsc.VectorSubcoreMesh(core_axis_name="c", subcore_axis_name="s")

    @functools.partial(
        pl.kernel,
        out_type=jax.ShapeDtypeStruct((HIST, EMBED_DIM, BATCH), jnp.float32),
        scratch_types=[
            pltpu.VMEM((128,), jnp.int32),
            pltpu.VMEM((128,), jnp.int32),
            pltpu.VMEM((128, EMBED_DIM), jnp.float32),
            pltpu.VMEM((128, EMBED_DIM), jnp.float32),
            pltpu.VMEM((EMBED_DIM, 128), jnp.float32),
            pltpu.VMEM((EMBED_DIM, 128), jnp.float32),
            pltpu.SemaphoreType.DMA,
            pltpu.SemaphoreType.DMA,
            pltpu.SemaphoreType.DMA,
            pltpu.SemaphoreType.DMA,
            pltpu.SemaphoreType.DMA,
            pltpu.SemaphoreType.DMA,
        ],
        mesh=mesh,
        compiler_params=pltpu.CompilerParams(use_tc_tiling_on_sc=False,
                                             needs_layout_passes=False),
    )
    def gather_kernel(idx_hbm, table_hbm, out_hbm,
                      idx0, idx1, g0, g1, t0, t1,
                      i0, i1, gs0, gs1, s0, s1):
        wid = lax.axis_index("s") * 2 + lax.axis_index("c")
        u_base = wid * _NU
        idx_v = (idx0, idx1)
        g_v = (g0, g1)
        t_v = (t0, t1)
        isem = (i0, i1)
        gsem = (gs0, gs1)
        ssem = (s0, s1)
        iota16 = lax.iota(jnp.int32, 16)

        def idx_slice(u):
            l = u // _JT
            j = u % _JT
            return idx_hbm.at[pl.ds(l * BATCH + j * 128, 128)]

        def out_slice(u):
            l = u // _JT
            j = u % _JT
            return out_hbm.at[l, :, pl.ds(j * 128, 128)]

        def issue_idx(b, u):
            pltpu.async_copy(idx_slice(u), idx_v[b], isem[b])

        def wait_idx(b, u):
            pltpu.make_async_copy(idx_slice(u), idx_v[b], isem[b]).wait()

        def issue_gather(b):
            pltpu.async_copy(table_hbm.at[idx_v[b]], g_v[b], gsem[b])

        def wait_gather(b):
            pltpu.make_async_copy(table_hbm.at[idx_v[b]], g_v[b],
                                  gsem[b]).wait()

        def issue_store(b, u):
            pltpu.async_copy(t_v[b], out_slice(u), ssem[b])

        def wait_store(b, u):
            pltpu.make_async_copy(t_v[b], out_slice(u), ssem[b]).wait()

        def transpose(b):
            gb = g_v[b]
            tb = t_v[b]

            def cbody(c, carry):
                col = jnp.full((16,), c, dtype=jnp.int32)
                for b0 in range(0, 128, 16):
                    vals = plsc.load_gather(gb, [iota16 + b0, col])
                    tb[c, pl.ds(b0, 16)] = vals
                return carry

            lax.fori_loop(0, EMBED_DIM, cbody, 0)

        issue_idx(0, u_base)
        wait_idx(0, u_base)
        issue_gather(0)
        issue_idx(1, u_base + 1)

        def body(p, carry):
            u0 = u_base + 2 * p
            u1 = u0 + 1

            wait_gather(0)
            wait_idx(1, u1)
            issue_gather(1)

            @pl.when(p < _NPAIR - 1)
            def _():
                issue_idx(0, u0 + 2)

            @pl.when(p > 0)
            def _():
                wait_store(0, u0 - 2)

            transpose(0)
            issue_store(0, u0)

            wait_gather(1)

            @pl.when(p < _NPAIR - 1)
            def _():
                wait_idx(0, u0 + 2)
                issue_gather(0)
                issue_idx(1, u1 + 2)

            @pl.when(p > 0)
            def _():
                wait_store(1, u1 - 2)

            transpose(1)
            issue_store(1, u1)
            return carry

        lax.fori_loop(0, _NPAIR, body, 0)
        wait_store(0, u_base + _NU - 2)
        wait_store(1, u_base + _NU - 1)

    return gather_kernel


_gather = _make_gather()


def kernel(x, table):
    idx = x.T.reshape(-1).astype(jnp.int32)
    out = _gather(idx, table)
    return out.transpose(2, 0, 1)


# speedup vs baseline: 1.6484x; 1.6484x over previous
import functools

import jax
import jax.numpy as jnp
from jax import lax
from jax.experimental import pallas as pl
from jax.experimental.pallas import tpu as pltpu
from jax.experimental.pallas import tpu_sc as plsc

VOCAB = 1000000
EMBED_DIM = 64
BATCH = 16384
HIST = 50

_NW = 32
_B_PER_W = BATCH // _NW
_BCH = 8
_CHUNK = _BCH * HIST
_NCH = _B_PER_W // _BCH
_NPAIR = _NCH // 2

_TSTEP = 2048
_TGRID = -(-VOCAB // _TSTEP)


def _detranspose_body(in_ref, out_ref):
    t = in_ref[...].T
    t3 = t.reshape(_TSTEP // 2, 2, EMBED_DIM)
    out_ref[...] = jnp.concatenate([t3[:, 0, :], t3[:, 1, :]], axis=-1)


_detranspose = pl.pallas_call(
    _detranspose_body,
    grid=(_TGRID,),
    in_specs=[pl.BlockSpec((EMBED_DIM, _TSTEP), lambda i: (0, i))],
    out_specs=pl.BlockSpec((_TSTEP // 2, 128), lambda i: (i, 0)),
    out_shape=jax.ShapeDtypeStruct((VOCAB // 2, 128), jnp.float32),
)


def _make_gather():
    mesh = plsc.VectorSubcoreMesh(core_axis_name="c", subcore_axis_name="s")

    @functools.partial(
        pl.kernel,
        out_type=jax.ShapeDtypeStruct((BATCH, HIST, EMBED_DIM), jnp.float32),
        scratch_types=[
            pltpu.VMEM((_CHUNK,), jnp.int32),
            pltpu.VMEM((_CHUNK,), jnp.int32),
            pltpu.VMEM((_CHUNK, EMBED_DIM), jnp.float32),
            pltpu.VMEM((_CHUNK, EMBED_DIM), jnp.float32),
            pltpu.SemaphoreType.DMA,
            pltpu.SemaphoreType.DMA,
            pltpu.SemaphoreType.DMA,
            pltpu.SemaphoreType.DMA,
        ],
        mesh=mesh,
        compiler_params=pltpu.CompilerParams(use_tc_tiling_on_sc=False),
    )
    def gather_kernel(idx_hbm, table_hbm, out_hbm,
                      idx0, idx1, rows0, rows1, g0, g1, s0, s1):
        wid = lax.axis_index("s") * 2 + lax.axis_index("c")
        base_b = wid * _B_PER_W
        idx_v = (idx0, idx1)
        rows_v = (rows0, rows1)
        gsem = (g0, g1)
        ssem = (s0, s1)

        def issue_gather(b, i):
            off = (base_b + i * _BCH) * HIST
            pltpu.sync_copy(idx_hbm.at[pl.ds(off, _CHUNK)], idx_v[b])
            pltpu.async_copy(table_hbm.at[idx_v[b]], rows_v[b], gsem[b])

        def wait_gather(b):
            pltpu.make_async_copy(table_hbm.at[idx_v[b]], rows_v[b],
                                  gsem[b]).wait()

        def issue_store(b, i):
            b0 = base_b + i * _BCH
            for k in range(_BCH):
                pltpu.async_copy(rows_v[b].at[pl.ds(k * HIST, HIST), :],
                                 out_hbm.at[b0 + k], ssem[b])

        def wait_store(b, i):
            b0 = base_b + i * _BCH
            for k in range(_BCH):
                pltpu.make_async_copy(rows_v[b].at[pl.ds(k * HIST, HIST), :],
                                      out_hbm.at[b0 + k], ssem[b]).wait()

        issue_gather(0, 0)

        def body(j, carry):
            i0 = 2 * j
            i1 = i0 + 1
            wait_gather(0)
            issue_store(0, i0)

            @pl.when(j > 0)
            def _():
                wait_store(1, i0 - 1)

            issue_gather(1, i1)
            wait_gather(1)
            issue_store(1, i1)
            wait_store(0, i0)

            @pl.when(j < _NPAIR - 1)
            def _():
                issue_gather(0, i0 + 2)

            return carry

        lax.fori_loop(0, _NPAIR, body, 0)
        wait_store(1, _NCH - 1)

    return gather_kernel


_gather = _make_gather()


def kernel(x, table):
    idx = x.reshape(-1).astype(jnp.int32)
    tbl = _detranspose(table.T).reshape(VOCAB, EMBED_DIM)
    return _gather(idx, tbl)
